# baseline (speedup 1.0000x reference)
import functools

import jax
import jax.numpy as jnp
from jax import lax
from jax.experimental import pallas as pl
from jax.experimental.pallas import tpu as pltpu
from jax.experimental.pallas import tpu_sc as plsc

N = 10000
E = 640000
DIM = 16

NP = 10112
NC = 2
NS = 16
NW = NC * NS
K = 128
EW = E // NW
CHUNKS = EW // K
TAIL = EW - CHUNKS * K
ROWS_PT = NP // NS
EBLK = 1024


def _dot(a, b):
    return lax.dot_general(a, b, (((1,), (0,)), ((), ())),
                           preferred_element_type=jnp.float32)


def _edge_c1_body(ea4_ref, w_ref, b_ref, c_ref):
    c_ref[...] = _dot(ea4_ref[...], w_ref[...]) + b_ref[...]


def _edge_c1(ea4, w4bd, bt):
    nblk = 125
    full = lambda shp: pl.BlockSpec(shp, lambda i: (0, 0))
    return pl.pallas_call(
        _edge_c1_body,
        grid=(nblk,),
        in_specs=[
            pl.BlockSpec((1280, 64), lambda i: (i, 0)),
            full((64, 128)), full((1, 128)),
        ],
        out_specs=pl.BlockSpec((1280, 128), lambda i: (i, 0)),
        out_shape=jax.ShapeDtypeStruct((E // 4, 128), jnp.float32),
    )(ea4, w4bd, bt.reshape(1, -1))


def _edge_c23_body(ea8_ref, w2_ref, b2_ref, w3_ref, b3_ref, c2_ref, c3_ref):
    ea8 = ea8_ref[...]
    c2_ref[...] = _dot(ea8, w2_ref[...]) + b2_ref[...]
    c3_ref[...] = _dot(ea8, w3_ref[...]) + b3_ref[...]


def _edge_c23(ea8, w2bd, b2t, w3bd, b3t):
    nblk = 125
    full = lambda shp: pl.BlockSpec(shp, lambda i: (0, 0))
    return pl.pallas_call(
        _edge_c23_body,
        grid=(nblk,),
        in_specs=[
            pl.BlockSpec((640, 128), lambda i: (i, 0)),
            full((128, 128)), full((1, 128)),
            full((128, 128)), full((1, 128)),
        ],
        out_specs=[
            pl.BlockSpec((640, 128), lambda i: (i, 0)),
            pl.BlockSpec((640, 128), lambda i: (i, 0)),
        ],
        out_shape=[
            jax.ShapeDtypeStruct((E // 8, 128), jnp.float32),
            jax.ShapeDtypeStruct((E // 8, 128), jnp.float32),
        ],
    )(ea8, w2bd, b2t.reshape(1, -1), w3bd, b3t.reshape(1, -1))


def _block_diag(w, copies):
    i, o = w.shape
    out = jnp.zeros((copies * i, copies * o), w.dtype)
    for q in range(copies):
        out = out.at[q * i:(q + 1) * i, q * o:(q + 1) * o].set(w)
    return out


def _block_diag(w, copies):
    i, o = w.shape
    out = jnp.zeros((copies * i, copies * o), w.dtype)
    for q in range(copies):
        out = out.at[q * i:(q + 1) * i, q * o:(q + 1) * o].set(w)
    return out


def _node1_body(h_ref, wd_ref, ws_ref, wu_ref, ub_ref, a_ref, b_ref, s_ref):
    h = h_ref[...]
    a_ref[...] = _dot(h, wd_ref[...])
    b_ref[...] = _dot(h, ws_ref[...])
    s_ref[...] = _dot(h, wu_ref[...]) + ub_ref[...]


def _node1(h1p, wd, ws, wu, ub):
    d = wd.shape[1]
    return pl.pallas_call(
        _node1_body,
        out_shape=[jax.ShapeDtypeStruct((NP, d), jnp.float32)] * 3,
    )(h1p, wd, ws, wu, ub.reshape(1, -1))


def _mid_body(agg_ref, sp_ref, wd_ref, ws_ref, wu_ref, ub_ref,
              a_ref, b_ref, s_ref):
    h = agg_ref[0:NP, :] + agg_ref[NP:2 * NP, :] + sp_ref[...]
    a_ref[...] = _dot(h, wd_ref[...])
    b_ref[...] = _dot(h, ws_ref[...])
    s_ref[...] = _dot(h, wu_ref[...]) + ub_ref[...]


def _mid(agg, s_prev, wd, ws, wu, ub):
    d = wd.shape[1]
    return pl.pallas_call(
        _mid_body,
        out_shape=[jax.ShapeDtypeStruct((NP, d), jnp.float32)] * 3,
    )(agg, s_prev, wd, ws, wu, ub.reshape(1, -1))


def _final_body(agg_ref, s_ref, o_ref):
    o_ref[...] = jnp.tanh(agg_ref[0:NP, :] + agg_ref[NP:2 * NP, :] + s_ref[...])


def _final(agg, s3):
    return pl.pallas_call(
        _final_body,
        out_shape=jax.ShapeDtypeStruct((NP, 16), jnp.float32),
    )(agg, s3)


def _make_sc_layer(d):
    mesh = plsc.VectorSubcoreMesh(core_axis_name="c", subcore_axis_name="s")
    cpr = 128 // d
    crows = K // cpr
    trows = TAIL // cpr

    @functools.partial(
        pl.kernel,
        mesh=mesh,
        compiler_params=pltpu.CompilerParams(use_tc_tiling_on_sc=False),
        out_type=jax.ShapeDtypeStruct((NC * NP, d), jnp.float32),
        scratch_types=[
            pltpu.VMEM((EW,), jnp.int32),
            pltpu.VMEM((EW,), jnp.int32),
            pltpu.VMEM((2, K), jnp.int32),
            pltpu.VMEM((2, K, d), jnp.float32),
            pltpu.VMEM((2, K, d), jnp.float32),
            pltpu.VMEM((2, crows, 128), jnp.float32),
            pltpu.VMEM((TAIL,), jnp.int32),
            pltpu.VMEM((TAIL, d), jnp.float32),
            pltpu.VMEM((TAIL, d), jnp.float32),
            pltpu.VMEM((trows, 128), jnp.float32),
            pltpu.VMEM_SHARED((NP, d), jnp.float32),
            pltpu.SemaphoreType.DMA,
            pltpu.SemaphoreType.DMA,
            pltpu.SemaphoreType.DMA,
        ],
    )
    def sc_layer(src_hbm, dst_hbm, zero_hbm, a_hbm, b_hbm, c_hbm, out_hbm,
                 srcv, dstv, dsts, av, bv, cv, tdst, tav, tbv, tcv, agg,
                 sem0, sem1, sem2):
        cid = lax.axis_index("c")
        sid = lax.axis_index("s")
        wid = sid * NC + cid
        ebase = wid * EW
        cbase = wid * (EW // cpr)
        row0 = sid * ROWS_PT
        sems = (sem0, sem1)

        pltpu.sync_copy(zero_hbm, agg.at[pl.ds(row0, ROWS_PT)])

        pltpu.sync_copy(src_hbm.at[pl.ds(ebase, EW)], srcv)
        pltpu.sync_copy(dst_hbm.at[pl.ds(ebase, EW)], dstv)

        plsc.subcore_barrier()

        for j in range(TAIL // 16):
            tdst[pl.ds(j * 16, 16)] = dstv[pl.ds(CHUNKS * K + j * 16, 16)]
        pltpu.async_copy(c_hbm.at[pl.ds(cbase + CHUNKS * crows, trows)],
                         tcv, sem2)
        pltpu.async_copy(a_hbm.at[tdst], tav, sem2)
        pltpu.async_copy(b_hbm.at[srcv.at[pl.ds(CHUNKS * K, TAIL)]], tbv, sem2)

        def _compute(a_r, b_r, c_r, nrows):
            def _row(rr, carry):
                for j in range(8):
                    f = j * 16
                    ar = rr * cpr + f // d
                    asl = pl.ds(f % d, 16)
                    v = a_r[ar, asl] + b_r[ar, asl] + c_r[rr, pl.ds(f, 16)]
                    a_r[ar, asl] = jnp.maximum(v, v * 0.01)
                return carry
            lax.fori_loop(0, nrows, _row, 0)

        def _issue(slot, g):
            for j in range(K // 16):
                dsts[slot, pl.ds(j * 16, 16)] = dstv[pl.ds(g * K + j * 16, 16)]
            pltpu.async_copy(c_hbm.at[pl.ds(cbase + g * crows, crows)],
                             cv.at[slot], sems[slot])
            pltpu.async_copy(a_hbm.at[dsts.at[slot]], av.at[slot], sems[slot])
            pltpu.async_copy(b_hbm.at[srcv.at[pl.ds(g * K, K)]],
                             bv.at[slot], sems[slot])

        def _process(slot):
            pltpu.make_async_copy(c_hbm.at[pl.ds(0, crows)],
                                  cv.at[slot], sems[slot]).wait()
            pltpu.make_async_copy(a_hbm.at[dsts.at[slot]],
                                  av.at[slot], sems[slot]).wait()
            pltpu.make_async_copy(b_hbm.at[srcv.at[pl.ds(0, K)]],
                                  bv.at[slot], sems[slot]).wait()
            _compute(av.at[slot], bv.at[slot], cv.at[slot], crows)
            pltpu.sync_copy(av.at[slot], agg.at[dsts.at[slot]], add=True)

        _issue(0, 0)
        _issue(1, 1)

        def _pair(it, carry):
            g = it * 2
            _process(0)
            _issue(0, g + 2)
            _process(1)
            _issue(1, g + 3)
            return carry

        lax.fori_loop(0, CHUNKS // 2 - 1, _pair, 0)
        _process(0)
        _process(1)

        pltpu.make_async_copy(c_hbm.at[pl.ds(0, trows)], tcv, sem2).wait()
        pltpu.make_async_copy(a_hbm.at[tdst], tav, sem2).wait()
        pltpu.make_async_copy(b_hbm.at[srcv.at[pl.ds(0, TAIL)]],
                              tbv, sem2).wait()
        _compute(tav, tbv, tcv, trows)
        pltpu.sync_copy(tav, agg.at[tdst], add=True)

        plsc.subcore_barrier()
        pltpu.sync_copy(agg.at[pl.ds(row0, ROWS_PT)],
                        out_hbm.at[pl.ds(cid * NP + row0, ROWS_PT)])

    return sc_layer


_sc32 = _make_sc_layer(32)
_sc16 = _make_sc_layer(16)


def kernel(x, edge_index, edge_attr, lower, upper,
           W1, b1, U1, ub1, W2, b2, U2, ub2, W3, b3, U3, ub3):
    f32 = jnp.float32
    src = edge_index[0]
    dst = edge_index[1]

    h1 = jnp.concatenate([x, lower, upper], axis=1)
    h1p = jnp.concatenate([h1, jnp.zeros((NP - N, 66), f32)], axis=0)

    W1d, W1s, W1e = W1[:, :66].T, W1[:, 66:132].T, W1[:, 132:].T
    W2d, W2s, W2e = W2[:, :32].T, W2[:, 32:64].T, W2[:, 64:].T
    W3p = jnp.pad(W3, ((0, 9), (0, 0)))
    W3d, W3s, W3e = W3p[:, :16].T, W3p[:, 16:32].T, W3p[:, 32:].T
    b3p = jnp.pad(b3, (0, 9))
    U3t = jnp.pad(U3, ((0, 9), (0, 0))).T
    ub3p = jnp.pad(ub3, (0, 9))

    zero32 = jnp.zeros((ROWS_PT, 32), f32)
    zero16 = jnp.zeros((ROWS_PT, 16), f32)

    ea4 = edge_attr.reshape(E // 4, 64)
    ea8 = edge_attr.reshape(E // 8, 128)
    C1 = _edge_c1(ea4, _block_diag(W1e, 4), jnp.tile(b1, 4))
    C2, C3 = _edge_c23(ea8, _block_diag(W2e, 8), jnp.tile(b2, 8),
                       _block_diag(W3e, 8), jnp.tile(b3p, 8))

    A1, B1, S1 = _node1(h1p, W1d, W1s, U1.T, ub1)
    agg1 = _sc32(src, dst, zero32, A1, B1, C1)
    A2, B2, S2 = _mid(agg1, S1, W2d, W2s, U2.T, ub2)
    agg2 = _sc16(src, dst, zero16, A2, B2, C2)
    A3, B3, S3 = _mid(agg2, S2, W3d, W3s, U3t, ub3p)
    agg3 = _sc16(src, dst, zero16, A3, B3, C3)
    out = _final(agg3, S3)
    return out[:N, :7]


# speedup vs baseline: 1.0539x; 1.0539x over previous
import functools

import jax
import jax.numpy as jnp
from jax import lax
from jax.experimental import pallas as pl
from jax.experimental.pallas import tpu as pltpu
from jax.experimental.pallas import tpu_sc as plsc

N = 10000
E = 640000
DIM = 16

NP = 10112
NC = 2
NS = 16
NW = NC * NS
K = 128
EW = E // NW
CHUNKS = EW // K
TAIL = EW - CHUNKS * K
ROWS_PT = NP // NS
EBLK = 1024


def _dot(a, b):
    return lax.dot_general(a, b, (((1,), (0,)), ((), ())),
                           preferred_element_type=jnp.float32)


def _edge_c1_body(ea4_ref, w_ref, b_ref, c_ref):
    c_ref[...] = _dot(ea4_ref[...], w_ref[...]) + b_ref[...]


def _edge_c1(ea4, w4bd, bt):
    nblk = 125
    full = lambda shp: pl.BlockSpec(shp, lambda i: (0, 0))
    return pl.pallas_call(
        _edge_c1_body,
        grid=(nblk,),
        in_specs=[
            pl.BlockSpec((1280, 64), lambda i: (i, 0)),
            full((64, 128)), full((1, 128)),
        ],
        out_specs=pl.BlockSpec((1280, 128), lambda i: (i, 0)),
        out_shape=jax.ShapeDtypeStruct((E // 4, 128), jnp.float32),
    )(ea4, w4bd, bt.reshape(1, -1))


def _block_diag(w, copies):
    i, o = w.shape
    out = jnp.zeros((copies * i, copies * o), w.dtype)
    for q in range(copies):
        out = out.at[q * i:(q + 1) * i, q * o:(q + 1) * o].set(w)
    return out


def _block_diag(w, copies):
    i, o = w.shape
    out = jnp.zeros((copies * i, copies * o), w.dtype)
    for q in range(copies):
        out = out.at[q * i:(q + 1) * i, q * o:(q + 1) * o].set(w)
    return out


def _node1_body(h_ref, wd_ref, ws_ref, wu_ref, ub_ref, a_ref, b_ref, s_ref):
    h = h_ref[...]
    a_ref[...] = _dot(h, wd_ref[...])
    b_ref[...] = _dot(h, ws_ref[...])
    s_ref[...] = _dot(h, wu_ref[...]) + ub_ref[...]


def _node1(h1p, wd, ws, wu, ub):
    d = wd.shape[1]
    return pl.pallas_call(
        _node1_body,
        out_shape=[jax.ShapeDtypeStruct((NP, d), jnp.float32)] * 3,
    )(h1p, wd, ws, wu, ub.reshape(1, -1))


def _mid_body(agg_ref, sp_ref, wd_ref, ws_ref, wu_ref, ub_ref,
              a_ref, b_ref, s_ref):
    h = agg_ref[0:NP, :] + agg_ref[NP:2 * NP, :] + sp_ref[...]
    a_ref[...] = _dot(h, wd_ref[...])
    b_ref[...] = _dot(h, ws_ref[...])
    s_ref[...] = _dot(h, wu_ref[...]) + ub_ref[...]


def _mid(agg, s_prev, wd, ws, wu, ub):
    d = wd.shape[1]
    return pl.pallas_call(
        _mid_body,
        out_shape=[jax.ShapeDtypeStruct((NP, d), jnp.float32)] * 3,
    )(agg, s_prev, wd, ws, wu, ub.reshape(1, -1))


def _final_body(agg_ref, s_ref, o_ref):
    o_ref[...] = jnp.tanh(agg_ref[0:NP, :] + agg_ref[NP:2 * NP, :] + s_ref[...])


def _final(agg, s3):
    return pl.pallas_call(
        _final_body,
        out_shape=jax.ShapeDtypeStruct((NP, 16), jnp.float32),
    )(agg, s3)


def _make_sc_layer(d, coff=0):
    mesh = plsc.VectorSubcoreMesh(core_axis_name="c", subcore_axis_name="s")
    crows = K // 4
    trows = TAIL // 4

    @functools.partial(
        pl.kernel,
        mesh=mesh,
        compiler_params=pltpu.CompilerParams(use_tc_tiling_on_sc=False),
        out_type=jax.ShapeDtypeStruct((NC * NP, d), jnp.float32),
        scratch_types=[
            pltpu.VMEM((EW,), jnp.int32),
            pltpu.VMEM((EW,), jnp.int32),
            pltpu.VMEM((2, K), jnp.int32),
            pltpu.VMEM((2, K, d), jnp.float32),
            pltpu.VMEM((2, K, d), jnp.float32),
            pltpu.VMEM((2, crows, 128), jnp.float32),
            pltpu.VMEM((TAIL,), jnp.int32),
            pltpu.VMEM((TAIL, d), jnp.float32),
            pltpu.VMEM((TAIL, d), jnp.float32),
            pltpu.VMEM((trows, 128), jnp.float32),
            pltpu.VMEM_SHARED((NP, d), jnp.float32),
            pltpu.SemaphoreType.DMA,
            pltpu.SemaphoreType.DMA,
            pltpu.SemaphoreType.DMA,
        ],
    )
    def sc_layer(src_hbm, dst_hbm, zero_hbm, a_hbm, b_hbm, c_hbm, out_hbm,
                 srcv, dstv, dsts, av, bv, cv, tdst, tav, tbv, tcv, agg,
                 sem0, sem1, sem2):
        cid = lax.axis_index("c")
        sid = lax.axis_index("s")
        wid = sid * NC + cid
        ebase = wid * EW
        cbase = wid * (EW // 4)
        row0 = sid * ROWS_PT
        sems = (sem0, sem1)

        pltpu.sync_copy(zero_hbm, agg.at[pl.ds(row0, ROWS_PT)])

        pltpu.sync_copy(src_hbm.at[pl.ds(ebase, EW)], srcv)
        pltpu.sync_copy(dst_hbm.at[pl.ds(ebase, EW)], dstv)

        plsc.subcore_barrier()

        for j in range(TAIL // 16):
            tdst[pl.ds(j * 16, 16)] = dstv[pl.ds(CHUNKS * K + j * 16, 16)]
        pltpu.async_copy(c_hbm.at[pl.ds(cbase + CHUNKS * crows, trows)],
                         tcv, sem2)
        pltpu.async_copy(a_hbm.at[tdst], tav, sem2)
        pltpu.async_copy(b_hbm.at[srcv.at[pl.ds(CHUNKS * K, TAIL)]], tbv, sem2)

        def _compute(a_r, b_r, c_r, nrows):
            def _row(rr, carry):
                if d == 32:
                    for j in range(8):
                        ar = rr * 4 + j // 2
                        asl = pl.ds((j % 2) * 16, 16)
                        v = (a_r[ar, asl] + b_r[ar, asl]
                             + c_r[rr, pl.ds(j * 16, 16)])
                        a_r[ar, asl] = jnp.maximum(v, v * 0.01)
                else:
                    for j in range(4):
                        ar = rr * 4 + j
                        asl = pl.ds(0, 16)
                        v = (a_r[ar, asl] + b_r[ar, asl]
                             + c_r[rr, pl.ds(j * 32 + coff, 16)])
                        a_r[ar, asl] = jnp.maximum(v, v * 0.01)
                return carry
            lax.fori_loop(0, nrows, _row, 0)

        def _issue(slot, g):
            for j in range(K // 16):
                dsts[slot, pl.ds(j * 16, 16)] = dstv[pl.ds(g * K + j * 16, 16)]
            pltpu.async_copy(c_hbm.at[pl.ds(cbase + g * crows, crows)],
                             cv.at[slot], sems[slot])
            pltpu.async_copy(a_hbm.at[dsts.at[slot]], av.at[slot], sems[slot])
            pltpu.async_copy(b_hbm.at[srcv.at[pl.ds(g * K, K)]],
                             bv.at[slot], sems[slot])

        def _process(slot):
            pltpu.make_async_copy(c_hbm.at[pl.ds(0, crows)],
                                  cv.at[slot], sems[slot]).wait()
            pltpu.make_async_copy(a_hbm.at[dsts.at[slot]],
                                  av.at[slot], sems[slot]).wait()
            pltpu.make_async_copy(b_hbm.at[srcv.at[pl.ds(0, K)]],
                                  bv.at[slot], sems[slot]).wait()
            _compute(av.at[slot], bv.at[slot], cv.at[slot], crows)
            pltpu.sync_copy(av.at[slot], agg.at[dsts.at[slot]], add=True)

        _issue(0, 0)
        _issue(1, 1)

        def _pair(it, carry):
            g = it * 2
            _process(0)
            _issue(0, g + 2)
            _process(1)
            _issue(1, g + 3)
            return carry

        lax.fori_loop(0, CHUNKS // 2 - 1, _pair, 0)
        _process(0)
        _process(1)

        pltpu.make_async_copy(c_hbm.at[pl.ds(0, trows)], tcv, sem2).wait()
        pltpu.make_async_copy(a_hbm.at[tdst], tav, sem2).wait()
        pltpu.make_async_copy(b_hbm.at[srcv.at[pl.ds(0, TAIL)]],
                              tbv, sem2).wait()
        _compute(tav, tbv, tcv, trows)
        pltpu.sync_copy(tav, agg.at[tdst], add=True)

        plsc.subcore_barrier()
        pltpu.sync_copy(agg.at[pl.ds(row0, ROWS_PT)],
                        out_hbm.at[pl.ds(cid * NP + row0, ROWS_PT)])

    return sc_layer


_sc32 = _make_sc_layer(32)
_sc16a = _make_sc_layer(16, coff=0)
_sc16b = _make_sc_layer(16, coff=16)


def kernel(x, edge_index, edge_attr, lower, upper,
           W1, b1, U1, ub1, W2, b2, U2, ub2, W3, b3, U3, ub3):
    f32 = jnp.float32
    src = edge_index[0]
    dst = edge_index[1]

    h1 = jnp.concatenate([x, lower, upper], axis=1)
    h1p = jnp.concatenate([h1, jnp.zeros((NP - N, 66), f32)], axis=0)

    W1d, W1s, W1e = W1[:, :66].T, W1[:, 66:132].T, W1[:, 132:].T
    W2d, W2s, W2e = W2[:, :32].T, W2[:, 32:64].T, W2[:, 64:].T
    W3p = jnp.pad(W3, ((0, 9), (0, 0)))
    W3d, W3s, W3e = W3p[:, :16].T, W3p[:, 16:32].T, W3p[:, 32:].T
    b3p = jnp.pad(b3, (0, 9))
    U3t = jnp.pad(U3, ((0, 9), (0, 0))).T
    ub3p = jnp.pad(ub3, (0, 9))

    zero32 = jnp.zeros((ROWS_PT, 32), f32)
    zero16 = jnp.zeros((ROWS_PT, 16), f32)

    ea4 = edge_attr.reshape(E // 4, 64)
    C1 = _edge_c1(ea4, _block_diag(W1e, 4), jnp.tile(b1, 4))
    C23 = _edge_c1(ea4, _block_diag(jnp.concatenate([W2e, W3e], axis=1), 4),
                   jnp.tile(jnp.concatenate([b2, b3p]), 4))

    A1, B1, S1 = _node1(h1p, W1d, W1s, U1.T, ub1)
    agg1 = _sc32(src, dst, zero32, A1, B1, C1)
    A2, B2, S2 = _mid(agg1, S1, W2d, W2s, U2.T, ub2)
    agg2 = _sc16a(src, dst, zero16, A2, B2, C23)
    A3, B3, S3 = _mid(agg2, S2, W3d, W3s, U3t, ub3p)
    agg3 = _sc16b(src, dst, zero16, A3, B3, C23)
    out = _final(agg3, S3)
    return out[:N, :7]
